# baseline (speedup 1.0000x reference)
import dataclasses
import functools

import jax
import jax.numpy as jnp
from jax import lax
from jax.experimental import pallas as pl
from jax.experimental.pallas import tpu as pltpu
from jax.experimental.pallas import tpu_sc as plsc

NUM_CORES = 2
NUM_SUBCORES = 16
NUM_WORKERS = NUM_CORES * NUM_SUBCORES
CHUNK = 128
LANES = 16
EPS = 1e-12


def _fused_sc(table, ids, gamma, beta, tokens, seq, hidden):
    batch, _ = ids.shape
    rows_per_worker = tokens // NUM_WORKERS
    nchunks = rows_per_worker // CHUNK
    workers_per_row = seq // rows_per_worker
    groups = hidden // LANES
    mesh = plsc.VectorSubcoreMesh(core_axis_name="c", subcore_axis_name="s")
    cparams = pltpu.CompilerParams()
    if "needs_layout_passes" in pltpu.CompilerParams.__dataclass_fields__:
        cparams = dataclasses.replace(cparams, needs_layout_passes=False)

    @functools.partial(
        pl.kernel,
        out_type=jax.ShapeDtypeStruct((tokens, hidden), jnp.float32),
        mesh=mesh,
        compiler_params=cparams,
        scratch_types=[
            pltpu.VMEM((rows_per_worker,), jnp.int32),
            pltpu.VMEM((nchunks, CHUNK, hidden), jnp.float32),
            pltpu.VMEM((nchunks, CHUNK, hidden), jnp.float32),
            pltpu.VMEM((hidden,), jnp.float32),
            pltpu.VMEM((hidden,), jnp.float32),
            pltpu.VMEM((CHUNK,), jnp.float32),
            pltpu.VMEM((CHUNK,), jnp.float32),
            pltpu.SemaphoreType.DMA,
            pltpu.SemaphoreType.DMA,
            pltpu.SemaphoreType.DMA,
            pltpu.SemaphoreType.DMA,
            pltpu.SemaphoreType.DMA,
            pltpu.SemaphoreType.DMA,
        ],
    )
    def fused_kernel(table_hbm, idx_hbm, gamma_hbm, beta_hbm, out_hbm,
                     idx_v, tok_v, pos_v, gm_v, bt_v, m_v, k_v,
                     sg0, sg1, sp0, sp1, so0, so1):
        wid = lax.axis_index("s") * NUM_CORES + lax.axis_index("c")
        base = wid * rows_per_worker
        brow = wid // workers_per_row
        col = (wid % workers_per_row) * rows_per_worker

        pltpu.sync_copy(gamma_hbm, gm_v)
        pltpu.sync_copy(beta_hbm, bt_v)
        pltpu.sync_copy(idx_hbm.at[brow, pl.ds(col, rows_per_worker)], idx_v)

        gsems = [sg0, sg1]
        psems = [sp0, sp1]
        osems = [so0, so1]
        gcp = []
        pcp = []
        for c in range(nchunks):
            gcp.append(
                pltpu.async_copy(
                    table_hbm.at[idx_v.at[pl.ds(c * CHUNK, CHUNK)]],
                    tok_v.at[c], gsems[c]))
            pos0 = (base + c * CHUNK) % seq
            pcp.append(
                pltpu.async_copy(
                    table_hbm.at[pl.ds(pos0, CHUNK)], pos_v.at[c], psems[c]))

        gs = [gm_v[pl.ds(i * LANES, LANES)] for i in range(groups)]
        bs = [bt_v[pl.ds(i * LANES, LANES)] for i in range(groups)]

        ocp = []
        for c in range(nchunks):
            gcp[c].wait()
            pcp[c].wait()

            @pl.loop(0, CHUNK // LANES)
            def _(g, c=c):
                r0 = g * LANES
                for u in range(LANES):
                    r = r0 + u
                    for i in range(groups):
                        sl = pl.ds(i * LANES, LANES)
                        tok_v[c, r, sl] = tok_v[c, r, sl] + pos_v[c, r, sl]
                row_idx = lax.iota(jnp.int32, LANES) + r0
                s1 = jnp.zeros((LANES,), jnp.float32)
                s2 = jnp.zeros((LANES,), jnp.float32)
                for j in range(hidden):
                    col_idx = jnp.full((LANES,), j, dtype=jnp.int32)
                    v = plsc.load_gather(tok_v.at[c], [row_idx, col_idx])
                    s1 = s1 + v
                    s2 = s2 + v * v
                m = s1 * (1.0 / hidden)
                q = s2 * (1.0 / hidden) - m * m + EPS
                yi = jnp.int32(0x5F3759DF) - (plsc.bitcast(q, jnp.int32) >> 1)
                y = plsc.bitcast(yi, jnp.float32)
                half = q * 0.5
                for _it in range(2):
                    y = y * (1.5 - half * y * y)
                m_v[pl.ds(r0, LANES)] = m
                k_v[pl.ds(r0, LANES)] = y
                for u in range(LANES):
                    r = r0 + u
                    ridx = jnp.full((LANES,), r, dtype=jnp.int32)
                    mv = plsc.load_gather(m_v, [ridx])
                    kv = plsc.load_gather(k_v, [ridx])
                    for i in range(groups):
                        sl = pl.ds(i * LANES, LANES)
                        tok_v[c, r, sl] = (
                            (tok_v[c, r, sl] - mv) * kv * gs[i] + bs[i])

            ocp.append(
                pltpu.async_copy(
                    tok_v.at[c], out_hbm.at[pl.ds(base + c * CHUNK, CHUNK)],
                    osems[c]))
        for cp in ocp:
            cp.wait()

    return fused_kernel(table, ids, gamma, beta)


@jax.jit
def _impl(input_ids, token_table, pos_table, ln_gamma, ln_beta):
    batch, seq = input_ids.shape
    hidden = token_table.shape[1]
    tokens = batch * seq
    out = _fused_sc(token_table, input_ids.astype(jnp.int32), ln_gamma,
                    ln_beta, tokens, seq, hidden)
    return out.reshape(batch, seq, hidden)


def kernel(input_ids, token_table, pos_table, ln_gamma, ln_beta):
    return _impl(input_ids, token_table, pos_table, ln_gamma, ln_beta)


# speedup vs baseline: 1.8752x; 1.8752x over previous
import functools

import jax
import jax.numpy as jnp
from jax import lax
from jax.experimental import pallas as pl
from jax.experimental.pallas import tpu as pltpu
from jax.experimental.pallas import tpu_sc as plsc

NUM_CORES = 2
NUM_SUBCORES = 16
NUM_WORKERS = NUM_CORES * NUM_SUBCORES
GATHER_CHUNK = 128

TC_BLOCK = 2048


def _sc_gather(table, ids, tokens, hidden):
    batch, seq = ids.shape
    rows_per_worker = tokens // NUM_WORKERS
    chunks = rows_per_worker // GATHER_CHUNK
    workers_per_row = seq // rows_per_worker
    mesh = plsc.VectorSubcoreMesh(core_axis_name="c", subcore_axis_name="s")

    @functools.partial(
        pl.kernel,
        out_type=jax.ShapeDtypeStruct((tokens, hidden), jnp.float32),
        mesh=mesh,
        scratch_types=[
            pltpu.VMEM((rows_per_worker,), jnp.int32),
            pltpu.VMEM((rows_per_worker, hidden), jnp.float32),
            pltpu.SemaphoreType.DMA,
        ],
    )
    def gather_kernel(table_hbm, idx_hbm, out_hbm, idx_v, rows_v, sem):
        wid = lax.axis_index("s") * NUM_CORES + lax.axis_index("c")
        base = wid * rows_per_worker
        b = wid // workers_per_row
        col = (wid % workers_per_row) * rows_per_worker
        pltpu.sync_copy(idx_hbm.at[b, pl.ds(col, rows_per_worker)], idx_v)
        copies = []
        for j in range(chunks):
            copies.append(
                pltpu.async_copy(
                    table_hbm.at[idx_v.at[pl.ds(j * GATHER_CHUNK, GATHER_CHUNK)]],
                    rows_v.at[pl.ds(j * GATHER_CHUNK, GATHER_CHUNK)],
                    sem,
                )
            )
        for cp in copies:
            cp.wait()
        pltpu.sync_copy(rows_v, out_hbm.at[pl.ds(base, rows_per_worker)])

    return gather_kernel(table, ids)


def _tc_add_ln(gathered, table, gamma, beta, tokens, seq, hidden):
    def body(g_ref, p_ref, gm_ref, bt_ref, o_ref):
        e = g_ref[...] + p_ref[...]
        m = jnp.mean(e, axis=1, keepdims=True)
        s2 = jnp.mean(e * e, axis=1, keepdims=True)
        k = lax.rsqrt(s2 - m * m + 1e-12)
        o_ref[...] = (e - m) * k * gm_ref[...] + bt_ref[...]

    pos_blocks = seq // TC_BLOCK
    batch = tokens // seq
    return pl.pallas_call(
        body,
        grid=(pos_blocks, batch),
        in_specs=[
            pl.BlockSpec((TC_BLOCK, hidden), lambda j, b: (b * pos_blocks + j, 0)),
            pl.BlockSpec((TC_BLOCK, hidden), lambda j, b: (j, 0)),
            pl.BlockSpec((1, hidden), lambda j, b: (0, 0)),
            pl.BlockSpec((1, hidden), lambda j, b: (0, 0)),
        ],
        out_specs=pl.BlockSpec((TC_BLOCK, hidden), lambda j, b: (b * pos_blocks + j, 0)),
        out_shape=jax.ShapeDtypeStruct((tokens, hidden), jnp.float32),
        input_output_aliases={0: 0},
    )(gathered, table, gamma.reshape(1, hidden), beta.reshape(1, hidden))


@jax.jit
def _impl(input_ids, token_table, pos_table, ln_gamma, ln_beta):
    batch, seq = input_ids.shape
    hidden = token_table.shape[1]
    tokens = batch * seq
    gathered = _sc_gather(token_table, input_ids.astype(jnp.int32), tokens, hidden)
    out = _tc_add_ln(gathered, token_table, ln_gamma, ln_beta, tokens, seq, hidden)
    return out.reshape(batch, seq, hidden)


def kernel(input_ids, token_table, pos_table, ln_gamma, ln_beta):
    return _impl(input_ids, token_table, pos_table, ln_gamma, ln_beta)
